# baseline (speedup 1.0000x reference)
import jax
import jax.numpy as jnp
from jax import lax
from jax.experimental import pallas as pl
from jax.experimental.pallas import tpu as pltpu
from jax.experimental.pallas import tpu_sc as plsc

_SEED_CNT = 128
_EMBED_DIM = 256
_MACHINE_CNT = 16
_LANES = 16
_NUM_WORKERS = 32
_BLOCK = _MACHINE_CNT * _EMBED_DIM


def _col_body(rand_hbm, col_hbm, rand_v, block0_v, block1_v, sems):
    rows_per_w = rand_hbm.shape[0] // _SEED_CNT // _NUM_WORKERS
    wid = lax.axis_index("s") * 2 + lax.axis_index("c")
    base = wid * rows_per_w
    pltpu.sync_copy(rand_hbm.at[pl.ds(base * _SEED_CNT, rows_per_w * _SEED_CNT)],
                    rand_v)

    iota = lax.iota(jnp.int32, _LANES)
    ones = jnp.ones((_LANES,), jnp.float32)
    zeros = jnp.zeros((_LANES,), jnp.float32)

    def zero_init(j, carry):
        blk = j // (_BLOCK // _LANES)
        jj = j % (_BLOCK // _LANES)
        b_ref = block0_v if blk == 0 else block1_v
        b_ref[jj // _LANES, pl.ds((jj % _LANES) * _LANES, _LANES)] = zeros
        return carry

    for j in range(2 * (_BLOCK // _LANES)):
        zero_init(j, 0)

    def sorted_idx(i):
        cur = None
        for j in range(_SEED_CNT // _LANES):
            v = rand_v[pl.ds(i * _SEED_CNT + j * _LANES, _LANES)]
            k = (v * 8388608.0).astype(jnp.int32) * _SEED_CNT + (iota + j * _LANES)
            k = plsc.bitcast(k, jnp.uint32)
            s, _ = plsc.sort_key_val(k, k)
            if cur is None:
                cur = s
            else:
                m = jnp.minimum(cur, lax.rev(s, (0,)))
                cur, _ = plsc.sort_key_val(m, m)
        return lax.bitwise_and(plsc.bitcast(cur, jnp.int32), _SEED_CNT - 1)

    def half_step(p, i, blk_ref, sem, offs_prev):
        idx = sorted_idx(i)

        @pl.when(p > 0)
        def _():
            pltpu.make_async_copy(blk_ref, col_hbm.at[base + i], sem).wait()

        plsc.store_scatter(blk_ref, [iota, offs_prev], zeros)
        plsc.store_scatter(blk_ref, [iota, idx], ones)
        pltpu.make_async_copy(blk_ref, col_hbm.at[base + i], sem).start()
        return idx

    def per_pair(p, carry):
        offs0_prev, offs1_prev = carry
        offs0 = half_step(p, 2 * p, block0_v, sems.at[0], offs0_prev)
        offs1 = half_step(p, 2 * p + 1, block1_v, sems.at[1], offs1_prev)
        return (offs0, offs1)

    lax.fori_loop(0, rows_per_w // 2, per_pair, (iota, iota))
    pltpu.make_async_copy(block0_v, col_hbm.at[base], sems.at[0]).wait()
    pltpu.make_async_copy(block1_v, col_hbm.at[base], sems.at[1]).wait()


def _make_col_kernel(batch_size):
    rows_per_w = batch_size // _NUM_WORKERS
    mesh = plsc.VectorSubcoreMesh(core_axis_name="c", subcore_axis_name="s")
    return pl.kernel(
        _col_body,
        out_type=jax.ShapeDtypeStruct((batch_size, _MACHINE_CNT, _EMBED_DIM),
                                      jnp.float32),
        mesh=mesh,
        compiler_params=pltpu.CompilerParams(needs_layout_passes=False,
                                             use_tc_tiling_on_sc=True,
                                             skip_device_barrier=True),
        scratch_types=[
            pltpu.VMEM((rows_per_w * _SEED_CNT,), jnp.float32),
            pltpu.VMEM((_MACHINE_CNT, _EMBED_DIM), jnp.float32),
            pltpu.VMEM((_MACHINE_CNT, _EMBED_DIM), jnp.float32),
            pltpu.SemaphoreType.DMA((2,)),
        ],
    )


def kernel(problems):
    batch_size, job_cnt, machine_cnt = problems.shape
    assert machine_cnt == _MACHINE_CNT and batch_size % _NUM_WORKERS == 0
    row_emb = jnp.zeros((batch_size, job_cnt, _EMBED_DIM), dtype=jnp.float32)
    rand_flat = jax.random.uniform(jax.random.key(42), (batch_size, _SEED_CNT),
                                   dtype=jnp.float32).reshape(-1)
    row_emb, rand_flat = lax.optimization_barrier((row_emb, rand_flat))
    col_emb = _make_col_kernel(batch_size)(rand_flat)
    return (row_emb, col_emb)


# speedup vs baseline: 1.0132x; 1.0132x over previous
import jax
import jax.numpy as jnp
from jax import lax
from jax.experimental import pallas as pl
from jax.experimental.pallas import tpu as pltpu
from jax.experimental.pallas import tpu_sc as plsc

_SEED_CNT = 128
_EMBED_DIM = 256
_MACHINE_CNT = 16
_LANES = 16
_NUM_WORKERS = 32
_BLOCK = _MACHINE_CNT * _EMBED_DIM


def _col_body(rand_hbm, col_hbm, rand_v, block0_v, block1_v, sems):
    rows_per_w = rand_hbm.shape[0] // _SEED_CNT // _NUM_WORKERS
    wid = lax.axis_index("s") * 2 + lax.axis_index("c")
    base = wid * rows_per_w
    pltpu.sync_copy(rand_hbm.at[pl.ds(base * _SEED_CNT, rows_per_w * _SEED_CNT)],
                    rand_v)

    iota = lax.iota(jnp.int32, _LANES)
    ones = jnp.ones((_LANES,), jnp.float32)
    zeros = jnp.zeros((_LANES,), jnp.float32)

    def zero_init(j, carry):
        blk = j // (_BLOCK // _LANES)
        jj = j % (_BLOCK // _LANES)
        b_ref = block0_v if blk == 0 else block1_v
        b_ref[jj // _LANES, pl.ds((jj % _LANES) * _LANES, _LANES)] = zeros
        return carry

    for j in range(2 * (_BLOCK // _LANES)):
        zero_init(j, 0)

    def sorted_idx(i):
        cur = None
        for j in range(_SEED_CNT // _LANES):
            v = rand_v[pl.ds(i * _SEED_CNT + j * _LANES, _LANES)]
            k = (v * 8388608.0).astype(jnp.int32) * _SEED_CNT + (iota + j * _LANES)
            k = plsc.bitcast(k, jnp.uint32)
            s, _ = plsc.sort_key_val(k, k)
            if cur is None:
                cur = s
            else:
                m = jnp.minimum(cur, lax.rev(s, (0,)))
                cur, _ = plsc.sort_key_val(m, m)
        return lax.bitwise_and(plsc.bitcast(cur, jnp.int32), _SEED_CNT - 1)

    def half_step(p, i, blk_ref, sem, offs_prev):
        idx = sorted_idx(i)

        @pl.when(p > 0)
        def _():
            pltpu.make_async_copy(blk_ref, col_hbm.at[base + i], sem).wait()

        plsc.store_scatter(blk_ref, [iota, offs_prev], zeros)
        plsc.store_scatter(blk_ref, [iota, idx], ones)
        pltpu.make_async_copy(blk_ref, col_hbm.at[base + i], sem).start()
        return idx

    def per_pair(p, carry):
        offs0_prev, offs1_prev = carry
        offs0 = half_step(p, 2 * p, block0_v, sems.at[0], offs0_prev)
        offs1 = half_step(p, 2 * p + 1, block1_v, sems.at[1], offs1_prev)
        return (offs0, offs1)

    lax.fori_loop(0, rows_per_w // 2, per_pair, (iota, iota))
    pltpu.make_async_copy(block0_v, col_hbm.at[base], sems.at[0]).wait()
    pltpu.make_async_copy(block1_v, col_hbm.at[base], sems.at[1]).wait()


def _make_col_kernel(batch_size):
    rows_per_w = batch_size // _NUM_WORKERS
    mesh = plsc.VectorSubcoreMesh(core_axis_name="c", subcore_axis_name="s")
    return pl.kernel(
        _col_body,
        out_type=jax.ShapeDtypeStruct((batch_size, _MACHINE_CNT, _EMBED_DIM),
                                      jnp.float32),
        mesh=mesh,
        compiler_params=pltpu.CompilerParams(needs_layout_passes=False,
                                             use_tc_tiling_on_sc=True),
        scratch_types=[
            pltpu.VMEM((rows_per_w * _SEED_CNT,), jnp.float32),
            pltpu.VMEM((_MACHINE_CNT, _EMBED_DIM), jnp.float32),
            pltpu.VMEM((_MACHINE_CNT, _EMBED_DIM), jnp.float32),
            pltpu.SemaphoreType.DMA((2,)),
        ],
    )


def kernel(problems):
    batch_size, job_cnt, machine_cnt = problems.shape
    assert machine_cnt == _MACHINE_CNT and batch_size % _NUM_WORKERS == 0
    row_emb = jnp.zeros((batch_size, job_cnt, _EMBED_DIM), dtype=jnp.float32)
    rand_flat = jax.random.uniform(jax.random.key(42), (batch_size, _SEED_CNT),
                                   dtype=jnp.float32).reshape(-1)
    row_emb, rand_flat = lax.optimization_barrier((row_emb, rand_flat))
    col_emb = _make_col_kernel(batch_size)(rand_flat)
    return (row_emb, col_emb)
